# baseline (speedup 1.0000x reference)
import jax
import jax.numpy as jnp
from jax import lax
from jax.experimental import pallas as pl
from jax.experimental.pallas import tpu as pltpu
from jax.experimental.pallas import tpu_sc as plsc

_B = 4096
_D = 64
_L = 20
_NC = 2
_NS = 16
_NW = _NC * _NS
_BPW = _B // _NW
_LANES = 16
_RING = 8


def _sc_embed_body(uidx_hbm, midx_hbm, utab_hbm, mtab_hbm, out_hbm,
                   uidx_v, midx_v, urows_v, mbuf_v, pooled_v,
                   usem, msems):
    wid = lax.axis_index("s") * _NC + lax.axis_index("c")
    ubase = pl.multiple_of(wid * _BPW, _BPW)
    pltpu.sync_copy(uidx_hbm.at[pl.ds(ubase, _BPW)], uidx_v)
    pltpu.sync_copy(midx_hbm.at[pl.ds(ubase, _BPW), :], midx_v)
    ucopy = pltpu.async_copy(utab_hbm.at[uidx_v], urows_v, usem)

    for b in range(_RING):
        pltpu.async_copy(
            mtab_hbm.at[midx_v.at[b]], mbuf_v.at[b], msems.at[b])

    def outer(g, carry):
        for b in range(_RING):
            r = g * _RING + b
            buf = mbuf_v.at[b]
            pltpu.make_async_copy(
                mtab_hbm.at[midx_v.at[r]], buf, msems.at[b]).wait()
            for dc in range(_D // _LANES):
                msl = pl.ds(dc * _LANES, _LANES)
                acc = buf[0, msl]
                for el in range(1, _L):
                    acc = acc + buf[el, msl]
                pooled_v[r, pl.ds(dc * _LANES, _LANES)] = acc
            nr = r + _RING
            @pl.when(nr < _BPW)
            def _():
                pltpu.async_copy(
                    mtab_hbm.at[midx_v.at[nr]], buf, msems.at[b])
        return carry

    lax.fori_loop(0, _BPW // _RING, outer, 0)
    ucopy.wait()

    def merge(r, carry):
        for dc in range(_D // _LANES):
            urows_v[r, pl.ds(_D + dc * _LANES, _LANES)] = (
                pooled_v[r, pl.ds(dc * _LANES, _LANES)])
        return carry

    lax.fori_loop(0, _BPW, merge, 0)
    pltpu.sync_copy(urows_v, out_hbm.at[pl.ds(ubase, _BPW), :])


def _sc_embed(uidx, midx, utab, mtab):
    mesh = plsc.VectorSubcoreMesh(
        core_axis_name="c", subcore_axis_name="s",
        num_cores=_NC, num_subcores=_NS)
    f = pl.kernel(
        _sc_embed_body,
        out_type=jax.ShapeDtypeStruct((_B, 2 * _D), jnp.float32),
        mesh=mesh,
        compiler_params=pltpu.CompilerParams(use_tc_tiling_on_sc=True),
        scratch_types=[
            pltpu.VMEM((_BPW,), jnp.int32),
            pltpu.VMEM((_BPW, _L), jnp.int32),
            pltpu.VMEM((_BPW, 2 * _D), jnp.float32),
            pltpu.VMEM((_RING, _L, 2 * _D), jnp.float32),
            pltpu.VMEM((_BPW, _D), jnp.float32),
            pltpu.SemaphoreType.DMA,
            pltpu.SemaphoreType.DMA((_RING,)),
        ],
    )
    return f(uidx, midx, utab, mtab)


_BT = 512


def _mlp_body(x_ref, w1_ref, b1_ref, w2_ref, b2_ref, w3_ref, b3_ref, o_ref):
    h = jnp.dot(x_ref[...], w1_ref[...], preferred_element_type=jnp.float32)
    h = jnp.maximum(h + b1_ref[...][None, :], 0.0)
    h = jnp.dot(h, w2_ref[...], preferred_element_type=jnp.float32)
    h = jnp.maximum(h + b2_ref[...][None, :], 0.0)
    o_ref[...] = jnp.dot(h, w3_ref[...], precision="highest",
                         preferred_element_type=jnp.float32) + b3_ref[...]


def _mlp(x, w1, b1, w2, b2, w3, b3):
    grid = (_B // _BT,)
    full2 = lambda i: (0, 0)
    full1 = lambda i: (0,)
    return pl.pallas_call(
        _mlp_body,
        grid=grid,
        in_specs=[
            pl.BlockSpec((_BT, 2 * _D), lambda i: (i, 0)),
            pl.BlockSpec((2 * _D, 256), full2),
            pl.BlockSpec((256,), full1),
            pl.BlockSpec((256, 128), full2),
            pl.BlockSpec((128,), full1),
            pl.BlockSpec((128,), full1),
            pl.BlockSpec((1,), full1),
        ],
        out_specs=pl.BlockSpec((_BT,), lambda i: (i,)),
        out_shape=jax.ShapeDtypeStruct((_B,), jnp.float32),
    )(x, w1, b1, w2, b2, w3, b3)


def kernel(user_indices, movie_indices, user_table, movie_table,
           W1, b1, W2, b2, W3, b3):
    uidx = user_indices.astype(jnp.int32)
    midx = movie_indices.astype(jnp.int32)
    utab = jnp.pad(user_table, ((0, 0), (0, _D)))
    mtab = jnp.pad(movie_table, ((0, 0), (0, _D)))
    emb = _sc_embed(uidx, midx, utab, mtab)
    return _mlp(emb, W1, b1, W2, b2, W3.reshape(-1), b3)


# speedup vs baseline: 1.0168x; 1.0168x over previous
import jax
import jax.numpy as jnp
from jax import lax
from jax.experimental import pallas as pl
from jax.experimental.pallas import tpu as pltpu
from jax.experimental.pallas import tpu_sc as plsc

_B = 4096
_D = 64
_L = 20
_NC = 2
_NS = 16
_NW = _NC * _NS
_BPW = _B // 2 // _NW
_LANES = 16
_RING = 8


def _sc_embed_body(half, uidx_hbm, midx_hbm, tab_hbm, out_hbm,
                   uidx_v, midx_v, urows_v, mbuf_v, pooled_v,
                   usem, msems):
    utab_hbm = mtab_hbm = tab_hbm
    wid = lax.axis_index("s") * _NC + lax.axis_index("c")
    ubase = pl.multiple_of(half * (_B // 2) + wid * _BPW, _BPW)
    pltpu.sync_copy(uidx_hbm.at[pl.ds(ubase, _BPW)], uidx_v)
    pltpu.sync_copy(midx_hbm.at[pl.ds(ubase, _BPW), :], midx_v)
    ucopy = pltpu.async_copy(utab_hbm.at[uidx_v], urows_v, usem)

    for b in range(_RING):
        pltpu.async_copy(
            mtab_hbm.at[midx_v.at[b]], mbuf_v.at[b], msems.at[b])

    def outer(g, carry):
        for b in range(_RING):
            r = g * _RING + b
            buf = mbuf_v.at[b]
            pltpu.make_async_copy(
                mtab_hbm.at[midx_v.at[r]], buf, msems.at[b]).wait()
            for dc in range(_D // _LANES):
                msl = pl.ds(_D + dc * _LANES, _LANES)
                acc = buf[0, msl]
                for el in range(1, _L):
                    acc = acc + buf[el, msl]
                pooled_v[r, pl.ds(dc * _LANES, _LANES)] = acc
            nr = r + _RING
            @pl.when(nr < _BPW)
            def _():
                pltpu.async_copy(
                    mtab_hbm.at[midx_v.at[nr]], buf, msems.at[b])
        return carry

    lax.fori_loop(0, _BPW // _RING, outer, 0)
    ucopy.wait()

    def merge(r, carry):
        for dc in range(_D // _LANES):
            urows_v[r, pl.ds(_D + dc * _LANES, _LANES)] = (
                pooled_v[r, pl.ds(dc * _LANES, _LANES)])
        return carry

    lax.fori_loop(0, _BPW, merge, 0)
    pltpu.sync_copy(
        urows_v,
        out_hbm.at[pl.ds(pl.multiple_of(wid * _BPW, _BPW), _BPW), :])


def _sc_embed(uidx, midx, table, half):
    import functools
    mesh = plsc.VectorSubcoreMesh(
        core_axis_name="c", subcore_axis_name="s",
        num_cores=_NC, num_subcores=_NS)
    f = pl.kernel(
        functools.partial(_sc_embed_body, half),
        out_type=jax.ShapeDtypeStruct((_B // 2, 2 * _D), jnp.float32),
        mesh=mesh,
        compiler_params=pltpu.CompilerParams(use_tc_tiling_on_sc=True),
        scratch_types=[
            pltpu.VMEM((_BPW,), jnp.int32),
            pltpu.VMEM((_BPW, _L), jnp.int32),
            pltpu.VMEM((_BPW, 2 * _D), jnp.float32),
            pltpu.VMEM((_RING, _L, 2 * _D), jnp.float32),
            pltpu.VMEM((_BPW, _D), jnp.float32),
            pltpu.SemaphoreType.DMA,
            pltpu.SemaphoreType.DMA((_RING,)),
        ],
    )
    return f(uidx, midx, table)


_BT = 512


def _mlp_body(x_ref, w1_ref, b1_ref, w2_ref, b2_ref, w3_ref, b3_ref, o_ref):
    h = jnp.dot(x_ref[...], w1_ref[...], preferred_element_type=jnp.float32)
    h = jnp.maximum(h + b1_ref[...][None, :], 0.0)
    h = jnp.dot(h, w2_ref[...], preferred_element_type=jnp.float32)
    h = jnp.maximum(h + b2_ref[...][None, :], 0.0)
    o_ref[...] = jnp.dot(h, w3_ref[...], precision="highest",
                         preferred_element_type=jnp.float32) + b3_ref[...]


def _mlp(x, w1, b1, w2, b2, w3, b3):
    n = x.shape[0]
    grid = (n // _BT,)
    full2 = lambda i: (0, 0)
    full1 = lambda i: (0,)
    return pl.pallas_call(
        _mlp_body,
        grid=grid,
        in_specs=[
            pl.BlockSpec((_BT, 2 * _D), lambda i: (i, 0)),
            pl.BlockSpec((2 * _D, 256), full2),
            pl.BlockSpec((256,), full1),
            pl.BlockSpec((256, 128), full2),
            pl.BlockSpec((128,), full1),
            pl.BlockSpec((128,), full1),
            pl.BlockSpec((1,), full1),
        ],
        out_specs=pl.BlockSpec((_BT,), lambda i: (i,)),
        out_shape=jax.ShapeDtypeStruct((n,), jnp.float32),
    )(x, w1, b1, w2, b2, w3, b3)


def kernel(user_indices, movie_indices, user_table, movie_table,
           W1, b1, W2, b2, W3, b3):
    uidx = user_indices.astype(jnp.int32)
    midx = movie_indices.astype(jnp.int32)
    table = jnp.concatenate([user_table, movie_table], axis=1)
    w3v = W3.reshape(-1)
    emb0 = _sc_embed(uidx, midx, table, 0)
    emb1 = _sc_embed(uidx, midx, table, 1)
    o0 = _mlp(emb0, W1, b1, W2, b2, w3v, b3)
    o1 = _mlp(emb1, W1, b1, W2, b2, w3v, b3)
    return jnp.concatenate([o0, o1])


# speedup vs baseline: 1.0628x; 1.0452x over previous
import jax
import jax.numpy as jnp
from jax import lax
from jax.experimental import pallas as pl
from jax.experimental.pallas import tpu as pltpu
from jax.experimental.pallas import tpu_sc as plsc

_B = 4096
_D = 64
_L = 20
_NC = 2
_NS = 16
_NW = _NC * _NS
_BPW = _B // _NW
_LANES = 16
_RING = 8


def _sc_embed_body(uidx_hbm, midx_hbm, tab_hbm, out_hbm,
                   uidx_v, midx_v, urows_v, mbuf_v, pooled_v,
                   usem, msems):
    utab_hbm = mtab_hbm = tab_hbm
    wid = lax.axis_index("s") * _NC + lax.axis_index("c")
    ubase = pl.multiple_of(wid * _BPW, _BPW)
    pltpu.sync_copy(uidx_hbm.at[pl.ds(ubase, _BPW)], uidx_v)
    pltpu.sync_copy(midx_hbm.at[pl.ds(ubase, _BPW), :], midx_v)
    ucopy = pltpu.async_copy(utab_hbm.at[uidx_v], urows_v, usem)

    for b in range(_RING):
        pltpu.async_copy(
            mtab_hbm.at[midx_v.at[b]], mbuf_v.at[b], msems.at[b])

    def outer(g, carry):
        for b in range(_RING):
            r = g * _RING + b
            buf = mbuf_v.at[b]
            pltpu.make_async_copy(
                mtab_hbm.at[midx_v.at[r]], buf, msems.at[b]).wait()
            for dc in range(_D // _LANES):
                msl = pl.ds(_D + dc * _LANES, _LANES)
                acc = buf[0, msl]
                for el in range(1, _L):
                    acc = acc + buf[el, msl]
                pooled_v[r, pl.ds(dc * _LANES, _LANES)] = acc
            nr = r + _RING
            @pl.when(nr < _BPW)
            def _():
                pltpu.async_copy(
                    mtab_hbm.at[midx_v.at[nr]], buf, msems.at[b])
        return carry

    lax.fori_loop(0, _BPW // _RING, outer, 0)
    ucopy.wait()

    def merge(r, carry):
        for dc in range(_D // _LANES):
            urows_v[r, pl.ds(_D + dc * _LANES, _LANES)] = (
                pooled_v[r, pl.ds(dc * _LANES, _LANES)])
        return carry

    lax.fori_loop(0, _BPW, merge, 0)
    pltpu.sync_copy(urows_v, out_hbm.at[pl.ds(ubase, _BPW), :])


def _sc_embed(uidx, midx, table):
    mesh = plsc.VectorSubcoreMesh(
        core_axis_name="c", subcore_axis_name="s",
        num_cores=_NC, num_subcores=_NS)
    f = pl.kernel(
        _sc_embed_body,
        out_type=jax.ShapeDtypeStruct((_B, 2 * _D), jnp.float32),
        mesh=mesh,
        compiler_params=pltpu.CompilerParams(use_tc_tiling_on_sc=True),
        scratch_types=[
            pltpu.VMEM((_BPW,), jnp.int32),
            pltpu.VMEM((_BPW, _L), jnp.int32),
            pltpu.VMEM((_BPW, 2 * _D), jnp.float32),
            pltpu.VMEM((_RING, _L, 2 * _D), jnp.float32),
            pltpu.VMEM((_BPW, _D), jnp.float32),
            pltpu.SemaphoreType.DMA,
            pltpu.SemaphoreType.DMA((_RING,)),
        ],
    )
    return f(uidx, midx, table)


_BT = 512


def _mlp_body(x_ref, w1_ref, b1_ref, w2_ref, b2_ref, w3_ref, b3_ref, o_ref):
    h = jnp.dot(x_ref[...], w1_ref[...], preferred_element_type=jnp.float32)
    h = jnp.maximum(h + b1_ref[...][None, :], 0.0)
    h = jnp.dot(h, w2_ref[...], preferred_element_type=jnp.float32)
    h = jnp.maximum(h + b2_ref[...][None, :], 0.0)
    o_ref[...] = jnp.dot(h, w3_ref[...], precision="highest",
                         preferred_element_type=jnp.float32) + b3_ref[...]


def _mlp(x, w1, b1, w2, b2, w3, b3):
    n = x.shape[0]
    grid = (n // _BT,)
    full2 = lambda i: (0, 0)
    full1 = lambda i: (0,)
    return pl.pallas_call(
        _mlp_body,
        grid=grid,
        in_specs=[
            pl.BlockSpec((_BT, 2 * _D), lambda i: (i, 0)),
            pl.BlockSpec((2 * _D, 256), full2),
            pl.BlockSpec((256,), full1),
            pl.BlockSpec((256, 128), full2),
            pl.BlockSpec((128,), full1),
            pl.BlockSpec((128,), full1),
            pl.BlockSpec((1,), full1),
        ],
        out_specs=pl.BlockSpec((_BT,), lambda i: (i,)),
        out_shape=jax.ShapeDtypeStruct((n,), jnp.float32),
    )(x, w1, b1, w2, b2, w3, b3)


def kernel(user_indices, movie_indices, user_table, movie_table,
           W1, b1, W2, b2, W3, b3):
    uidx = user_indices.astype(jnp.int32)
    midx = movie_indices.astype(jnp.int32)
    table = jnp.concatenate([user_table, movie_table], axis=1)
    emb = _sc_embed(uidx, midx, table)
    return _mlp(emb, W1, b1, W2, b2, W3.reshape(-1), b3)
